# baseline (speedup 1.0000x reference)
import ml_dtypes
import numpy as np
import jax
import jax.numpy as jnp
from jax.experimental import pallas as pl
from jax.experimental.pallas import tpu as pltpu

_KAPPA = 30.0
_NLAT = 64
_NLON = 128


def _grid_np(nlat, nlon):
    lat = (np.arange(nlat) + 0.5) / nlat * np.pi - np.pi / 2.0
    lon = np.arange(nlon) / nlon * 2.0 * np.pi
    lat2, lon2 = np.meshgrid(lat, lon, indexing="ij")
    return np.stack(
        [
            (np.cos(lat2) * np.cos(lon2)).ravel(),
            (np.cos(lat2) * np.sin(lon2)).ravel(),
            np.sin(lat2).ravel(),
        ],
        axis=1,
    )


def _bf16(a):
    return a.astype(np.float32).astype(ml_dtypes.bfloat16).astype(np.float64)


def _vmf_np(nlat_in, nlon_in, nlat_out, nlon_out, kappa):
    v_in = _bf16(_grid_np(nlat_in, nlon_in))
    v_out = _bf16(_grid_np(nlat_out, nlon_out))
    logits = kappa * (v_out @ v_in.T)
    logits -= logits.max(axis=1, keepdims=True)
    w = np.exp(logits)
    w /= w.sum(axis=1, keepdims=True)
    return w


_K1 = _vmf_np(_NLAT, _NLON, _NLAT // 2, _NLON // 2, _KAPPA)
_K2 = _vmf_np(_NLAT // 2, _NLON // 2, _NLAT // 4, _NLON // 4, _KAPPA)
_KC = _K2 @ _K1

_R = 256
_U, _s, _Vt = np.linalg.svd(_KC, full_matrices=False)
_V = np.ascontiguousarray(_Vt[:_R].T).astype(ml_dtypes.bfloat16)
_UR = np.ascontiguousarray((_U[:, :_R] * _s[:_R]).T).astype(ml_dtypes.bfloat16)
_K2T = np.ascontiguousarray(_K2.T).astype(ml_dtypes.bfloat16)

_N_IN = _NLAT * _NLON
_N1 = _K2T.shape[0]
_N2 = _K2T.shape[1]
_KBLK = 4096
_NSTEPS = _N_IN // _KBLK


def _body(x_ref, v_ref, ur_ref, b1_ref, k2t_ref, b2_ref, o_ref, acc_ref,
          bc_ref):
    @pl.when(pl.program_id(1) == 0)
    def _init():
        acc_ref[...] = jnp.zeros_like(acc_ref)
        bc_ref[...] = jnp.dot(
            b1_ref[...].astype(jnp.bfloat16), k2t_ref[...],
            preferred_element_type=jnp.float32,
        ) + b2_ref[...]

    acc_ref[...] += jnp.dot(
        x_ref[...].astype(jnp.bfloat16), v_ref[...],
        preferred_element_type=jnp.float32,
    )

    @pl.when(pl.program_id(1) == _NSTEPS - 1)
    def _final():
        o_ref[...] = jnp.dot(
            acc_ref[...].astype(jnp.bfloat16), ur_ref[...],
            preferred_element_type=jnp.float32,
        ) + bc_ref[...]


def kernel(x, bias1, bias2):
    b, c, n_in = x.shape
    m = b * c
    xf = x.reshape(m, n_in)
    mh = m // 2
    out = pl.pallas_call(
        _body,
        grid=(2, _NSTEPS),
        in_specs=[
            pl.BlockSpec((mh, _KBLK), lambda i, k: (i, k)),
            pl.BlockSpec((_KBLK, _R), lambda i, k: (k, 0)),
            pl.BlockSpec((_R, _N2), lambda i, k: (0, 0)),
            pl.BlockSpec((1, _N1), lambda i, k: (0, 0)),
            pl.BlockSpec((_N1, _N2), lambda i, k: (0, 0)),
            pl.BlockSpec((1, _N2), lambda i, k: (0, 0)),
        ],
        out_specs=pl.BlockSpec((mh, _N2), lambda i, k: (i, 0)),
        out_shape=jax.ShapeDtypeStruct((m, _N2), jnp.float32),
        scratch_shapes=[pltpu.VMEM((mh, _R), jnp.float32),
                        pltpu.VMEM((1, _N2), jnp.float32)],
        compiler_params=pltpu.CompilerParams(
            dimension_semantics=("parallel", "arbitrary")
        ),
    )(xf, jnp.asarray(_V), jnp.asarray(_UR), bias1.reshape(1, _N1),
      jnp.asarray(_K2T), bias2.reshape(1, _N2))
    return out.reshape(b, c, _N2)


# speedup vs baseline: 1.2269x; 1.2269x over previous
import ml_dtypes
import numpy as np
import jax
import jax.numpy as jnp
from jax.experimental import pallas as pl
from jax.experimental.pallas import tpu as pltpu

_KAPPA = 30.0
_NLAT = 64
_NLON = 128


def _grid_np(nlat, nlon):
    lat = (np.arange(nlat) + 0.5) / nlat * np.pi - np.pi / 2.0
    lon = np.arange(nlon) / nlon * 2.0 * np.pi
    lat2, lon2 = np.meshgrid(lat, lon, indexing="ij")
    return np.stack(
        [
            (np.cos(lat2) * np.cos(lon2)).ravel(),
            (np.cos(lat2) * np.sin(lon2)).ravel(),
            np.sin(lat2).ravel(),
        ],
        axis=1,
    )


def _bf16(a):
    return a.astype(np.float32).astype(ml_dtypes.bfloat16).astype(np.float64)


def _vmf_np(nlat_in, nlon_in, nlat_out, nlon_out, kappa):
    v_in = _bf16(_grid_np(nlat_in, nlon_in))
    v_out = _bf16(_grid_np(nlat_out, nlon_out))
    logits = kappa * (v_out @ v_in.T)
    logits -= logits.max(axis=1, keepdims=True)
    w = np.exp(logits)
    w /= w.sum(axis=1, keepdims=True)
    return w


_K1 = _vmf_np(_NLAT, _NLON, _NLAT // 2, _NLON // 2, _KAPPA)
_K2 = _vmf_np(_NLAT // 2, _NLON // 2, _NLAT // 4, _NLON // 4, _KAPPA)
_KC = _K2 @ _K1

_R = 256
_U, _s, _Vt = np.linalg.svd(_KC, full_matrices=False)
_V = np.ascontiguousarray(_Vt[:_R].T).astype(ml_dtypes.bfloat16)
_UR = np.ascontiguousarray((_U[:, :_R] * _s[:_R]).T).astype(ml_dtypes.bfloat16)
_K2T = np.ascontiguousarray(_K2.T).astype(ml_dtypes.bfloat16)

_N_IN = _NLAT * _NLON
_N1 = _K2T.shape[0]
_N2 = _K2T.shape[1]
_KBLK = 4096
_NSTEPS = _N_IN // _KBLK


def _body(x_ref, v_ref, ur_ref, b1_ref, k2t_ref, b2_ref, o_ref, acc_ref,
          bc_ref):
    @pl.when(pl.program_id(0) == 0)
    def _init():
        acc_ref[...] = jnp.zeros_like(acc_ref)
        bc_ref[...] = b2_ref[...]

    acc_ref[...] += jnp.dot(
        x_ref[...].astype(jnp.bfloat16), v_ref[...],
        preferred_element_type=jnp.float32,
    )
    bc_ref[...] += jnp.dot(
        b1_ref[...].astype(jnp.bfloat16), k2t_ref[...],
        preferred_element_type=jnp.float32,
    )

    @pl.when(pl.program_id(0) == _NSTEPS - 1)
    def _final():
        o_ref[...] = jnp.dot(
            acc_ref[...].astype(jnp.bfloat16), ur_ref[...],
            preferred_element_type=jnp.float32,
        ) + bc_ref[...]


def kernel(x, bias1, bias2):
    b, c, n_in = x.shape
    m = b * c
    xf = x.reshape(m, n_in)
    n1blk = _N1 // _NSTEPS
    out = pl.pallas_call(
        _body,
        grid=(_NSTEPS,),
        in_specs=[
            pl.BlockSpec((m, _KBLK), lambda k: (0, k)),
            pl.BlockSpec((_KBLK, _R), lambda k: (k, 0)),
            pl.BlockSpec((_R, _N2), lambda k: (0, 0)),
            pl.BlockSpec((1, n1blk), lambda k: (0, k)),
            pl.BlockSpec((n1blk, _N2), lambda k: (k, 0)),
            pl.BlockSpec((1, _N2), lambda k: (0, 0)),
        ],
        out_specs=pl.BlockSpec((m, _N2), lambda k: (0, 0)),
        out_shape=jax.ShapeDtypeStruct((m, _N2), jnp.float32),
        scratch_shapes=[pltpu.VMEM((m, _R), jnp.float32),
                        pltpu.VMEM((1, _N2), jnp.float32)],
        compiler_params=pltpu.CompilerParams(
            dimension_semantics=("arbitrary",)
        ),
    )(xf, jnp.asarray(_V), jnp.asarray(_UR), bias1.reshape(1, _N1),
      jnp.asarray(_K2T), bias2.reshape(1, _N2))
    return out.reshape(b, c, _N2)


# speedup vs baseline: 1.2624x; 1.0289x over previous
import ml_dtypes
import numpy as np
import jax
import jax.numpy as jnp
from jax.experimental import pallas as pl
from jax.experimental.pallas import tpu as pltpu

_KAPPA = 30.0
_NLAT = 64
_NLON = 128


def _grid_np(nlat, nlon):
    lat = (np.arange(nlat) + 0.5) / nlat * np.pi - np.pi / 2.0
    lon = np.arange(nlon) / nlon * 2.0 * np.pi
    lat2, lon2 = np.meshgrid(lat, lon, indexing="ij")
    return np.stack(
        [
            (np.cos(lat2) * np.cos(lon2)).ravel(),
            (np.cos(lat2) * np.sin(lon2)).ravel(),
            np.sin(lat2).ravel(),
        ],
        axis=1,
    )


def _bf16(a):
    return a.astype(np.float32).astype(ml_dtypes.bfloat16).astype(np.float64)


def _vmf_np(nlat_in, nlon_in, nlat_out, nlon_out, kappa):
    v_in = _bf16(_grid_np(nlat_in, nlon_in))
    v_out = _bf16(_grid_np(nlat_out, nlon_out))
    logits = kappa * (v_out @ v_in.T)
    logits -= logits.max(axis=1, keepdims=True)
    w = np.exp(logits)
    w /= w.sum(axis=1, keepdims=True)
    return w


_K1 = _vmf_np(_NLAT, _NLON, _NLAT // 2, _NLON // 2, _KAPPA)
_K2 = _vmf_np(_NLAT // 2, _NLON // 2, _NLAT // 4, _NLON // 4, _KAPPA)
_KC = _K2 @ _K1

_R = 256
_U, _s, _Vt = np.linalg.svd(_KC, full_matrices=False)
_V = np.ascontiguousarray(_Vt[:_R].T).astype(ml_dtypes.bfloat16)
_UR = np.ascontiguousarray((_U[:, :_R] * _s[:_R]).T).astype(ml_dtypes.bfloat16)
_K2T = np.ascontiguousarray(_K2.T).astype(ml_dtypes.bfloat16)

_N_IN = _NLAT * _NLON
_N1 = _K2T.shape[0]
_N2 = _K2T.shape[1]
_KBLK = 4096
_NSTEPS = _N_IN // _KBLK


def _body(x_ref, v_ref, ur_ref, b1_ref, k2t_ref, b2_ref, o_ref, acc_ref,
          bc_ref):
    @pl.when(pl.program_id(0) == 0)
    def _init():
        acc_ref[...] = jnp.zeros_like(acc_ref)
        bc_ref[...] = jnp.dot(
            b1_ref[...].astype(jnp.bfloat16), k2t_ref[...],
            preferred_element_type=jnp.float32,
        ) + b2_ref[...]

    acc_ref[...] += jnp.dot(
        x_ref[...].astype(jnp.bfloat16), v_ref[...],
        preferred_element_type=jnp.float32,
    )

    @pl.when(pl.program_id(0) == _NSTEPS - 1)
    def _final():
        o_ref[...] = jnp.dot(
            acc_ref[...].astype(jnp.bfloat16), ur_ref[...],
            preferred_element_type=jnp.float32,
        ) + bc_ref[...]


def kernel(x, bias1, bias2):
    b, c, n_in = x.shape
    m = b * c
    xf = x.reshape(m, n_in)
    out = pl.pallas_call(
        _body,
        grid=(_NSTEPS,),
        in_specs=[
            pl.BlockSpec((m, _KBLK), lambda k: (0, k)),
            pl.BlockSpec((_KBLK, _R), lambda k: (k, 0)),
            pl.BlockSpec((_R, _N2), lambda k: (0, 0)),
            pl.BlockSpec((1, _N1), lambda k: (0, 0)),
            pl.BlockSpec((_N1, _N2), lambda k: (0, 0)),
            pl.BlockSpec((1, _N2), lambda k: (0, 0)),
        ],
        out_specs=pl.BlockSpec((m, _N2), lambda k: (0, 0)),
        out_shape=jax.ShapeDtypeStruct((m, _N2), jnp.float32),
        scratch_shapes=[pltpu.VMEM((m, _R), jnp.float32),
                        pltpu.VMEM((1, _N2), jnp.float32)],
        compiler_params=pltpu.CompilerParams(
            dimension_semantics=("arbitrary",)
        ),
    )(xf, jnp.asarray(_V), jnp.asarray(_UR), bias1.reshape(1, _N1),
      jnp.asarray(_K2T), bias2.reshape(1, _N2))
    return out.reshape(b, c, _N2)
